# baseline (speedup 1.0000x reference)
import functools

import jax
import jax.numpy as jnp
from jax.experimental import pallas as pl

_N_NODES = 10000
_N_EDGES = 160000
_N_ANGLES = 320000
_DIM = 128


def _edge_stage_body(dsrc_ref, edst_ref, ce_ref, bsrc_ref, enew_ref, sig_ref, sigb_ref):
    e_ji = dsrc_ref[...] + edst_ref[...] + ce_ref[...]
    sig = jax.nn.sigmoid(e_ji)
    enew_ref[...] = jax.nn.relu(e_ji)
    sig_ref[...] = sig
    sigb_ref[...] = sig * bsrc_ref[...]


def _edge_stage(dh_src, eh_dst, ce, bh_src, block=2000):
    n = dh_src.shape[0]
    grid = (n // block,)
    spec = pl.BlockSpec((block, _DIM), lambda i: (i, 0))
    out_shape = [jax.ShapeDtypeStruct((n, _DIM), jnp.float32)] * 3
    return pl.pallas_call(
        _edge_stage_body,
        grid=grid,
        in_specs=[spec, spec, spec, spec],
        out_specs=[spec, spec, spec],
        out_shape=out_shape,
    )(dh_src, eh_dst, ce, bh_src)


def _node_stage_body(ah_ref, num_ref, den_ref, h_ref):
    h_ref[...] = jax.nn.relu(ah_ref[...] + num_ref[...] / (den_ref[...] + 1e-6))


def _node_stage(ah, num, den, block=2000):
    n = ah.shape[0]
    grid = (n // block,)
    spec = pl.BlockSpec((block, _DIM), lambda i: (i, 0))
    return pl.pallas_call(
        _node_stage_body,
        grid=grid,
        in_specs=[spec, spec, spec],
        out_specs=spec,
        out_shape=jax.ShapeDtypeStruct((n, _DIM), jnp.float32),
    )(ah, num, den)


def _gated_layer(h, e, edge_index, p, n_nodes, node_block):
    src = edge_index[0]
    dst = edge_index[1]
    Ah = h @ p['A'][0] + p['A'][1]
    Bh = h @ p['B'][0] + p['B'][1]
    Dh = h @ p['D'][0] + p['D'][1]
    Eh = h @ p['E'][0] + p['E'][1]
    Ce = e @ p['C'][0] + p['C'][1]
    e_new, sig, sigb = _edge_stage(Dh[src], Eh[dst], Ce, Bh[src])
    num = jax.ops.segment_sum(sigb, dst, num_segments=n_nodes)
    den = jax.ops.segment_sum(sig, dst, num_segments=n_nodes)
    h_new = _node_stage(Ah, num, den, block=node_block)
    return h_new, e_new


def kernel(node_feats, edge_feats, angle_feats, graph_edge_index, line_graph_edge_index, params):
    h, e = _gated_layer(node_feats, edge_feats, graph_edge_index,
                        params['node_update'], _N_NODES, node_block=2000)
    e, a = _gated_layer(e, angle_feats, line_graph_edge_index,
                        params['edge_update'], _N_EDGES, node_block=2000)
    return (h, e, a)
